# baseline (speedup 1.0000x reference)
import functools

import jax
import jax.numpy as jnp
from jax import lax
from jax.experimental import pallas as pl
from jax.experimental.pallas import tpu as pltpu
from jax.experimental.pallas import tpu_sc as plsc

BATCH = 16384
LATENT = 128
OUT_W = 2 * LATENT + 1

NC, NS, L = 2, 16, 16
NW = NC * NS
ROWS_PER_W = BATCH // NW
CHUNK = 128
N_CHUNKS = ROWS_PER_W // CHUNK

_mesh = plsc.VectorSubcoreMesh(core_axis_name="c", subcore_axis_name="s")


@functools.partial(
    pl.kernel,
    out_type=(
        jax.ShapeDtypeStruct((BATCH,), jnp.float32),
        jax.ShapeDtypeStruct((BATCH, OUT_W), jnp.float32),
    ),
    mesh=_mesh,
    scratch_types=[
        pltpu.VMEM((3 * ROWS_PER_W,), jnp.int32),
        pltpu.VMEM((N_CHUNKS, CHUNK), jnp.int32),
        pltpu.VMEM((N_CHUNKS, CHUNK), jnp.int32),
        pltpu.VMEM((N_CHUNKS, CHUNK), jnp.float32),
        pltpu.VMEM((2 * CHUNK, LATENT), jnp.float32),
        pltpu.VMEM((2 * CHUNK, LATENT), jnp.float32),
        pltpu.VMEM((CHUNK, OUT_W), jnp.float32),
        pltpu.VMEM((L * L,), jnp.float32),
        pltpu.VMEM((CHUNK,), jnp.float32),
        pltpu.SemaphoreType.DMA,
        pltpu.SemaphoreType.DMA,
        pltpu.SemaphoreType.DMA,
    ],
    compiler_params=pltpu.CompilerParams(needs_layout_passes=False),
)
def _mf_kernel(inp_hbm, ut_hbm, it_hbm,
               rating_out, emb_out,
               inpblk, idx_u, idx_i, nrbuf, urows, irows, outblk, tbuf, dots,
               sem_u, sem_i, sem_w):
    wid = lax.axis_index("s") * NC + lax.axis_index("c")
    lanes = jnp.arange(L, dtype=jnp.int32)
    lanes3 = lanes * 3

    pltpu.sync_copy(inp_hbm.at[pl.ds(3 * wid * ROWS_PER_W, 3 * ROWS_PER_W)],
                    inpblk)
    for ci in range(N_CHUNKS):
        for rr in range(CHUNK // L):
            off = 3 * (ci * CHUNK + rr * L)
            idx_u[ci, pl.ds(rr * L, L)] = plsc.load_gather(
                inpblk, [lanes3 + off])
            idx_i[ci, pl.ds(rr * L, L)] = plsc.load_gather(
                inpblk, [lanes3 + (off + 1)])
            rat = plsc.load_gather(inpblk, [lanes3 + (off + 2)])
            nrbuf[ci, pl.ds(rr * L, L)] = rat.astype(jnp.float32) / 5.0

    def start_gathers(ci, p):
        cu = pltpu.async_copy(ut_hbm.at[idx_u.at[ci]],
                              urows.at[pl.ds(p * CHUNK, CHUNK), :], sem_u)
        cv = pltpu.async_copy(it_hbm.at[idx_i.at[ci]],
                              irows.at[pl.ds(p * CHUNK, CHUNK), :], sem_i)
        return cu, cv

    pend = start_gathers(0, 0)
    pend_w = None

    for ci in range(N_CHUNKS):
        p = ci & 1
        base = wid * ROWS_PER_W + ci * CHUNK
        pend[0].wait()
        pend[1].wait()
        if ci + 1 < N_CHUNKS:
            pend = start_gathers(ci + 1, 1 - p)
        if pend_w is not None:
            pend_w[0].wait()
            pend_w[1].wait()

        def group_body(g, _):
            row0 = g * L
            nr = nrbuf[ci, pl.ds(row0, L)]
            for rr in range(L):
                r = row0 + rr
                rbuf = p * CHUNK + r
                acc = None
                for k in range(LATENT // L):
                    u = urows[rbuf, pl.ds(k * L, L)]
                    v = irows[rbuf, pl.ds(k * L, L)]
                    outblk[r, pl.ds(k * L, L)] = u
                    outblk[r, pl.ds(LATENT + k * L, L)] = v
                    acc = u * v if acc is None else acc + u * v
                plsc.store_scatter(tbuf, [lanes * L + rr], acc)
            rows = row0 + lanes
            plsc.store_scatter(outblk, [rows, jnp.full((L,), 2 * LATENT)], nr)
            tot = tbuf[pl.ds(0, L)]
            for k in range(1, L):
                tot = tot + tbuf[pl.ds(k * L, L)]
            dots[pl.ds(row0, L)] = tot
            return 0

        lax.fori_loop(0, CHUNK // L, group_body, 0)

        cw1 = pltpu.async_copy(dots, rating_out.at[pl.ds(base, CHUNK)], sem_w)
        cw2 = pltpu.async_copy(outblk, emb_out.at[pl.ds(base, CHUNK), :], sem_w)
        pend_w = (cw1, cw2)

    pend_w[0].wait()
    pend_w[1].wait()


def kernel(inputs, user_table, item_table):
    rating_vec, embedded = _mf_kernel(inputs.reshape(-1),
                                      user_table, item_table)
    return rating_vec.reshape(-1, 1), embedded


# speedup vs baseline: 1.1107x; 1.1107x over previous
import functools

import jax
import jax.numpy as jnp
from jax import lax
from jax.experimental import pallas as pl
from jax.experimental.pallas import tpu as pltpu
from jax.experimental.pallas import tpu_sc as plsc

BATCH = 16384
LATENT = 128
OUT_W = 2 * LATENT + 1

NC, NS, L = 2, 16, 16
NW = NC * NS
ROWS_PER_W = BATCH // NW
CHUNK = 128
N_CHUNKS = ROWS_PER_W // CHUNK

_mesh = plsc.VectorSubcoreMesh(core_axis_name="c", subcore_axis_name="s")


@functools.partial(
    pl.kernel,
    out_type=(
        jax.ShapeDtypeStruct((BATCH,), jnp.float32),
        jax.ShapeDtypeStruct((BATCH, OUT_W), jnp.float32),
    ),
    mesh=_mesh,
    scratch_types=[
        pltpu.VMEM((N_CHUNKS, CHUNK), jnp.int32),
        pltpu.VMEM((N_CHUNKS, CHUNK), jnp.int32),
        pltpu.VMEM((N_CHUNKS, CHUNK), jnp.int32),
        pltpu.VMEM((N_CHUNKS, CHUNK), jnp.float32),
        pltpu.VMEM((2 * CHUNK, LATENT), jnp.float32),
        pltpu.VMEM((2 * CHUNK, LATENT), jnp.float32),
        pltpu.VMEM((CHUNK, OUT_W), jnp.float32),
        pltpu.VMEM((L * L,), jnp.float32),
        pltpu.VMEM((CHUNK,), jnp.float32),
        pltpu.SemaphoreType.DMA,
        pltpu.SemaphoreType.DMA,
        pltpu.SemaphoreType.DMA,
    ],
    compiler_params=pltpu.CompilerParams(needs_layout_passes=False),
)
def _mf_kernel(uid_hbm, iid_hbm, rat_hbm, ut_hbm, it_hbm,
               rating_out, emb_out,
               idx_u, idx_i, ratbuf, nrbuf, urows, irows, outblk, tbuf, dots,
               sem_u, sem_i, sem_w):
    wid = lax.axis_index("s") * NC + lax.axis_index("c")
    lanes = jnp.arange(L, dtype=jnp.int32)

    staged = []
    for ci in range(N_CHUNKS):
        cbase = wid * ROWS_PER_W + ci * CHUNK
        for src, dst in ((uid_hbm, idx_u), (iid_hbm, idx_i), (rat_hbm, ratbuf)):
            staged.append(pltpu.async_copy(src.at[pl.ds(cbase, CHUNK)],
                                           dst.at[ci], sem_w))
    for c in staged:
        c.wait()
    for ci in range(N_CHUNKS):
        for rr in range(CHUNK // L):
            r = ratbuf[ci, pl.ds(rr * L, L)]
            nrbuf[ci, pl.ds(rr * L, L)] = r.astype(jnp.float32) / 5.0

    def start_gathers(ci, p):
        cu = pltpu.async_copy(ut_hbm.at[idx_u.at[ci]],
                              urows.at[pl.ds(p * CHUNK, CHUNK), :], sem_u)
        cv = pltpu.async_copy(it_hbm.at[idx_i.at[ci]],
                              irows.at[pl.ds(p * CHUNK, CHUNK), :], sem_i)
        return cu, cv

    pend = start_gathers(0, 0)
    pend_w = None

    for ci in range(N_CHUNKS):
        p = ci & 1
        base = wid * ROWS_PER_W + ci * CHUNK
        pend[0].wait()
        pend[1].wait()
        if ci + 1 < N_CHUNKS:
            pend = start_gathers(ci + 1, 1 - p)
        if pend_w is not None:
            pend_w[0].wait()
            pend_w[1].wait()

        def group_body(g, _):
            row0 = g * L
            nr = nrbuf[ci, pl.ds(row0, L)]
            for rr in range(L):
                r = row0 + rr
                rbuf = p * CHUNK + r
                acc = None
                for k in range(LATENT // L):
                    u = urows[rbuf, pl.ds(k * L, L)]
                    v = irows[rbuf, pl.ds(k * L, L)]
                    outblk[r, pl.ds(k * L, L)] = u
                    outblk[r, pl.ds(LATENT + k * L, L)] = v
                    acc = u * v if acc is None else acc + u * v
                plsc.store_scatter(tbuf, [lanes * L + rr], acc)
            rows = row0 + lanes
            plsc.store_scatter(outblk, [rows, jnp.full((L,), 2 * LATENT)], nr)
            tot = tbuf[pl.ds(0, L)]
            for k in range(1, L):
                tot = tot + tbuf[pl.ds(k * L, L)]
            dots[pl.ds(row0, L)] = tot
            return 0

        lax.fori_loop(0, CHUNK // L, group_body, 0)

        cw1 = pltpu.async_copy(dots, rating_out.at[pl.ds(base, CHUNK)], sem_w)
        cw2 = pltpu.async_copy(outblk, emb_out.at[pl.ds(base, CHUNK), :], sem_w)
        pend_w = (cw1, cw2)

    pend_w[0].wait()
    pend_w[1].wait()


def kernel(inputs, user_table, item_table):
    uid = inputs[:, 0]
    iid = inputs[:, 1]
    rat = inputs[:, 2]
    rating_vec, embedded = _mf_kernel(uid, iid, rat, user_table, item_table)
    return rating_vec.reshape(-1, 1), embedded


# speedup vs baseline: 1.3489x; 1.2145x over previous
import functools

import jax
import jax.numpy as jnp
from jax import lax
from jax.experimental import pallas as pl
from jax.experimental.pallas import tpu as pltpu
from jax.experimental.pallas import tpu_sc as plsc

BATCH = 16384
LATENT = 128
OUT_W = 2 * LATENT + 1
ITEM_W = LATENT + 1

NC, NS, L = 2, 16, 16
NW = NC * NS
ROWS_PER_W = BATCH // NW
CHUNK = 128
N_CHUNKS = ROWS_PER_W // CHUNK

_mesh = plsc.VectorSubcoreMesh(core_axis_name="c", subcore_axis_name="s")


@functools.partial(
    pl.kernel,
    out_type=(
        jax.ShapeDtypeStruct((BATCH,), jnp.float32),
        jax.ShapeDtypeStruct((BATCH, OUT_W), jnp.float32),
    ),
    mesh=_mesh,
    scratch_types=[
        pltpu.VMEM((N_CHUNKS, CHUNK), jnp.int32),
        pltpu.VMEM((N_CHUNKS, CHUNK), jnp.int32),
        pltpu.VMEM((N_CHUNKS, CHUNK), jnp.int32),
        pltpu.VMEM((N_CHUNKS, CHUNK), jnp.float32),
        pltpu.VMEM((2 * CHUNK, LATENT), jnp.float32),
        pltpu.VMEM((2 * CHUNK, ITEM_W), jnp.float32),
        pltpu.VMEM((L * L,), jnp.float32),
        pltpu.VMEM((2, CHUNK), jnp.float32),
        pltpu.SemaphoreType.DMA,
        pltpu.SemaphoreType.DMA,
        pltpu.SemaphoreType.DMA,
    ],
    compiler_params=pltpu.CompilerParams(needs_layout_passes=False),
)
def _mf_kernel(uid_hbm, iid_hbm, rat_hbm, ut_hbm, it_hbm,
               rating_out, emb_out,
               idx_u, idx_i, ratbuf, nrbuf, ublk, vblk, tbuf, dots,
               sem_u, sem_i, sem_w):
    wid = lax.axis_index("s") * NC + lax.axis_index("c")
    lanes = jnp.arange(L, dtype=jnp.int32)

    staged = []
    for ci in range(N_CHUNKS):
        cbase = wid * ROWS_PER_W + ci * CHUNK
        for src, dst in ((uid_hbm, idx_u), (iid_hbm, idx_i), (rat_hbm, ratbuf)):
            staged.append(pltpu.async_copy(src.at[pl.ds(cbase, CHUNK)],
                                           dst.at[ci], sem_w))
    for c in staged:
        c.wait()
    for ci in range(N_CHUNKS):
        for rr in range(CHUNK // L):
            r = ratbuf[ci, pl.ds(rr * L, L)]
            nrbuf[ci, pl.ds(rr * L, L)] = r.astype(jnp.float32) / 5.0

    def start_gathers(ci, p):
        cu = pltpu.async_copy(ut_hbm.at[idx_u.at[ci]],
                              ublk.at[pl.ds(p * CHUNK, CHUNK), :], sem_u)
        cv = pltpu.async_copy(it_hbm.at[idx_i.at[ci]],
                              vblk.at[pl.ds(p * CHUNK, CHUNK), pl.ds(0, LATENT)],
                              sem_i)
        return cu, cv

    pend = start_gathers(0, 0)
    pend_w = None

    for ci in range(N_CHUNKS):
        p = ci & 1
        base = wid * ROWS_PER_W + ci * CHUNK
        pend[0].wait()
        pend[1].wait()
        if pend_w is not None:
            for c in pend_w:
                c.wait()
        if ci + 1 < N_CHUNKS:
            pend = start_gathers(ci + 1, 1 - p)

        def group_body(g, _):
            row0 = g * L
            nr = nrbuf[ci, pl.ds(row0, L)]
            for rr in range(L):
                rbuf = p * CHUNK + row0 + rr
                acc = None
                for k in range(LATENT // L):
                    u = ublk[rbuf, pl.ds(k * L, L)]
                    v = vblk[rbuf, pl.ds(k * L, L)]
                    acc = u * v if acc is None else acc + u * v
                plsc.store_scatter(tbuf, [lanes * L + rr], acc)
            rows = p * CHUNK + row0 + lanes
            plsc.store_scatter(vblk, [rows, jnp.full((L,), LATENT)], nr)
            tot = tbuf[pl.ds(0, L)]
            for k in range(1, L):
                tot = tot + tbuf[pl.ds(k * L, L)]
            dots[p, pl.ds(row0, L)] = tot
            return 0

        lax.fori_loop(0, CHUNK // L, group_body, 0)

        cw1 = pltpu.async_copy(dots.at[p], rating_out.at[pl.ds(base, CHUNK)],
                               sem_w)
        cw2 = pltpu.async_copy(ublk.at[pl.ds(p * CHUNK, CHUNK), :],
                               emb_out.at[pl.ds(base, CHUNK), pl.ds(0, LATENT)],
                               sem_w)
        cw3 = pltpu.async_copy(
            vblk.at[pl.ds(p * CHUNK, CHUNK), :],
            emb_out.at[pl.ds(base, CHUNK), pl.ds(LATENT, ITEM_W)], sem_w)
        pend_w = (cw1, cw2, cw3)

    for c in pend_w:
        c.wait()


def kernel(inputs, user_table, item_table):
    uid = inputs[:, 0]
    iid = inputs[:, 1]
    rat = inputs[:, 2]
    rating_vec, embedded = _mf_kernel(uid, iid, rat, user_table, item_table)
    return rating_vec.reshape(-1, 1), embedded
